# Initial kernel scaffold; baseline (speedup 1.0000x reference)
import functools

import jax
import jax.numpy as jnp
from jax import lax
from jax.experimental import pallas as pl
from jax.experimental.pallas import tpu as pltpu
from jax.experimental.pallas import tpu_sc as plsc

D = 32
NUM_CODES = 8192
NUM_TOKENS = 8192
TBLK = 512
CBLK = 1024

NC = 2
NS = 16
NW = NC * NS
BPW = NUM_TOKENS // NW
LANES = 16


def _argmin_body(x_ref, embt_ref, idx_ref):
    x = x_ref[...]
    best_m = jnp.full((TBLK,), jnp.inf, dtype=jnp.float32)
    best_i = jnp.zeros((TBLK,), dtype=jnp.int32)
    for k in range(NUM_CODES // CBLK):
        e = embt_ref[:, k * CBLK:(k + 1) * CBLK]
        e2 = jnp.sum(e * e, axis=0)
        sc = e2[None, :] - 2.0 * jnp.dot(
            x, e, preferred_element_type=jnp.float32)
        m = jnp.min(sc, axis=1)
        lane = lax.broadcasted_iota(jnp.int32, (TBLK, CBLK), 1)
        a = jnp.min(jnp.where(sc == m[:, None], lane, NUM_CODES), axis=1)
        better = m < best_m
        best_i = jnp.where(better, a + k * CBLK, best_i)
        best_m = jnp.where(better, m, best_m)
    idx_ref[...] = best_i.reshape(TBLK // 128, 128)


def _tc_argmin(flat_x, embt):
    grid = NUM_TOKENS // TBLK
    idx2d = pl.pallas_call(
        _argmin_body,
        grid=(grid,),
        in_specs=[
            pl.BlockSpec((TBLK, D), lambda i: (i, 0)),
            pl.BlockSpec((D, NUM_CODES), lambda i: (0, 0)),
        ],
        out_specs=pl.BlockSpec((TBLK // 128, 128), lambda i: (i, 0)),
        out_shape=jax.ShapeDtypeStruct((NUM_TOKENS // 128, 128), jnp.int32),
    )(flat_x, embt)
    return idx2d.reshape(-1)


@functools.partial(
    pl.kernel,
    out_type=(
        jax.ShapeDtypeStruct((NUM_TOKENS, D), jnp.float32),
        jax.ShapeDtypeStruct((NW, LANES), jnp.float32),
    ),
    mesh=plsc.VectorSubcoreMesh(core_axis_name="c", subcore_axis_name="s"),
    scratch_types=[
        pltpu.VMEM((BPW,), jnp.int32),
        pltpu.VMEM((BPW, D), jnp.float32),
        pltpu.VMEM((BPW, D), jnp.float32),
        pltpu.VMEM((LANES,), jnp.float32),
        pltpu.SemaphoreType.DMA,
    ],
)
def _sc_decode(emb_hbm, idx_hbm, x_hbm, q_hbm, part_hbm,
               idx_v, rows_v, x_v, acc_v, sem):
    wid = lax.axis_index("s") * NC + lax.axis_index("c")
    base = wid * BPW
    pltpu.sync_copy(idx_hbm.at[pl.ds(base, BPW)], idx_v)
    pltpu.sync_copy(x_hbm.at[pl.ds(base, BPW)], x_v)
    pltpu.async_copy(emb_hbm.at[idx_v], rows_v, sem).wait()
    acc_v[...] = jnp.zeros((LANES,), jnp.float32)

    def body(i, carry):
        for h in range(D // LANES):
            s = pl.ds(h * LANES, LANES)
            q = rows_v[i, s]
            xv = x_v[i, s]
            qq = xv + (q - xv)
            r = qq - xv
            acc_v[...] += r * r
            rows_v[i, s] = qq
        return carry

    lax.fori_loop(0, BPW, body, 0)
    pltpu.sync_copy(rows_v, q_hbm.at[pl.ds(base, BPW)])
    pltpu.sync_copy(acc_v, part_hbm.at[wid])


def kernel(x, embedding_weight):
    flat_x = x.reshape(-1, D)
    embt = embedding_weight.T
    idx = _tc_argmin(flat_x, embt)
    q, part = _sc_decode(embedding_weight, idx, flat_x)
    quantized = q.reshape(x.shape)
    diff = jnp.sum(part) / (NUM_TOKENS * D)
    return quantized, diff


# speedup vs baseline: 1.0337x; 1.0337x over previous
import functools

import jax
import jax.numpy as jnp
from jax import lax
from jax.experimental import pallas as pl
from jax.experimental.pallas import tpu as pltpu
from jax.experimental.pallas import tpu_sc as plsc

D = 32
NUM_CODES = 8192
NUM_TOKENS = 8192
TBLK = 512
CBLK = 2048

NC = 2
NS = 16
NW = NC * NS
BPW = NUM_TOKENS // NW
LANES = 16


def _argmin_body(x_ref, x2_ref, embt_ref, idx_ref):
    x = x_ref[...]
    x2 = x2_ref[...]
    xb = x.astype(jnp.bfloat16)
    run_m = jnp.full((TBLK,), jnp.inf, dtype=jnp.float32)
    best_i = jnp.zeros((TBLK,), dtype=jnp.int32)
    for k in range(NUM_CODES // CBLK):
        e = embt_ref[:, k * CBLK:(k + 1) * CBLK]
        e2 = jnp.sum(e * e, axis=0)
        mm = jnp.dot(xb, e.astype(jnp.bfloat16),
                     preferred_element_type=jnp.float32)
        sc = (x2 + e2[None, :]) - 2.0 * mm
        m = jnp.min(sc, axis=1)
        lane = lax.broadcasted_iota(jnp.int32, (TBLK, CBLK), 1)
        a = jnp.min(jnp.where(sc == m[:, None], lane, NUM_CODES), axis=1)
        better = m < run_m
        best_i = jnp.where(better, a + k * CBLK, best_i)
        m_st = m.astype(jnp.bfloat16).astype(jnp.float32)
        run_m = jnp.where(better, m_st, run_m)
    idx_ref[...] = best_i.reshape(1, TBLK // 128, 128)


def _tc_argmin(flat_x, x2, embt):
    grid = NUM_TOKENS // TBLK
    idx2d = pl.pallas_call(
        _argmin_body,
        grid=(grid,),
        in_specs=[
            pl.BlockSpec((TBLK, D), lambda i: (i, 0)),
            pl.BlockSpec((TBLK, 1), lambda i: (i, 0)),
            pl.BlockSpec((D, NUM_CODES), lambda i: (0, 0)),
        ],
        out_specs=pl.BlockSpec((1, TBLK // 128, 128), lambda i: (i, 0, 0)),
        out_shape=jax.ShapeDtypeStruct((grid, TBLK // 128, 128), jnp.int32),
    )(flat_x, x2, embt)
    return idx2d.reshape(-1)


def _sc_decode_body(emb_hbm, idx_hbm, x_hbm, q_hbm, part_hbm,
                    idx_v, rows_v, x_v, out_v, acc_v, sem):
    wid = lax.axis_index("s") * NC + lax.axis_index("c")
    base = wid * BPW
    pltpu.sync_copy(idx_hbm.at[pl.ds(base, BPW)], idx_v)
    pltpu.sync_copy(x_hbm.at[pl.ds(base, BPW)], x_v)
    copies = [
        pltpu.make_async_copy(
            emb_hbm.at[idx_v.at[pl.ds(j * 128, 128)]],
            rows_v.at[pl.ds(j * 128, 128)],
            sem,
        )
        for j in range(BPW // 128)
    ]
    for c in copies:
        c.start()
    for c in copies:
        c.wait()
    acc_v[...] = jnp.zeros((LANES,), jnp.float32)

    def body(i, carry):
        for h in range(D // LANES):
            s = pl.ds(h * LANES, LANES)
            q = rows_v[i, s]
            xv = x_v[i, s]
            qq = xv + (q - xv)
            r = qq - xv
            acc_v[...] += r * r
            out_v[i, s] = qq
        return carry

    lax.fori_loop(0, BPW, body, 0)
    pltpu.sync_copy(out_v, q_hbm.at[pl.ds(base, BPW)])
    pltpu.sync_copy(acc_v, part_hbm.at[wid])


@functools.cache
def _sc_decode():
    return pl.kernel(
        _sc_decode_body,
        out_type=(
            jax.ShapeDtypeStruct((NUM_TOKENS, D), jnp.float32),
            jax.ShapeDtypeStruct((NW, LANES), jnp.float32),
        ),
        mesh=plsc.VectorSubcoreMesh(
            core_axis_name="c", subcore_axis_name="s",
            num_cores=NC, num_subcores=NS),
        scratch_types=[
            pltpu.VMEM((BPW,), jnp.int32),
            pltpu.VMEM((BPW, 128), jnp.float32),
            pltpu.VMEM((BPW, D), jnp.float32),
            pltpu.VMEM((BPW, D), jnp.float32),
            pltpu.VMEM((LANES,), jnp.float32),
            pltpu.SemaphoreType.DMA,
        ],
    )


def kernel(x, embedding_weight):
    flat_x = x.reshape(-1, D)
    embt = embedding_weight.T
    x2 = jnp.sum(flat_x ** 2, axis=1, keepdims=True)
    idx = _tc_argmin(flat_x, x2, embt)
    emb_pad = jnp.pad(embedding_weight, ((0, 0), (0, 128 - D)))
    q, part = _sc_decode()(emb_pad, idx, flat_x)
    quantized = q.reshape(x.shape)
    diff = jnp.sum(part) / (NUM_TOKENS * D)
    return quantized, diff


# speedup vs baseline: 1.0832x; 1.0479x over previous
import functools

import jax
import jax.numpy as jnp
from jax import lax
from jax.experimental import pallas as pl
from jax.experimental.pallas import tpu as pltpu
from jax.experimental.pallas import tpu_sc as plsc

D = 32
NUM_CODES = 8192
NUM_TOKENS = 8192
TBLK = 512
CBLK = 2048

NC = 2
NS = 16
NW = NC * NS
BPW = NUM_TOKENS // NW
LANES = 16


def _argmin_body(x_ref, x2_ref, embt_ref, idx_ref):
    x = x_ref[...]
    x2 = x2_ref[...]
    xb2 = (x_ref[...].astype(jnp.bfloat16) * jnp.bfloat16(2.0))
    run_m = jnp.full((TBLK,), jnp.inf, dtype=jnp.float32)
    best_i = jnp.zeros((TBLK,), dtype=jnp.int32)
    for k in range(NUM_CODES // CBLK):
        e = embt_ref[:, k * CBLK:(k + 1) * CBLK]
        e2 = jnp.sum(e * e, axis=0)
        mm2 = jnp.dot(xb2, e.astype(jnp.bfloat16),
                      preferred_element_type=jnp.float32)
        sc = (x2 + e2[None, :]) - mm2
        m = jnp.min(sc, axis=1)
        a = jnp.argmin(sc, axis=1).astype(jnp.int32)
        better = m < run_m
        best_i = jnp.where(better, a + k * CBLK, best_i)
        m_st = m.astype(jnp.bfloat16).astype(jnp.float32)
        run_m = jnp.where(better, m_st, run_m)
    idx_ref[...] = best_i.reshape(1, TBLK // 128, 128)


def _tc_argmin(flat_x, x2, embt):
    grid = NUM_TOKENS // TBLK
    idx2d = pl.pallas_call(
        _argmin_body,
        grid=(grid,),
        in_specs=[
            pl.BlockSpec((TBLK, D), lambda i: (i, 0)),
            pl.BlockSpec((TBLK, 1), lambda i: (i, 0)),
            pl.BlockSpec((D, NUM_CODES), lambda i: (0, 0)),
        ],
        out_specs=pl.BlockSpec((1, TBLK // 128, 128), lambda i: (i, 0, 0)),
        out_shape=jax.ShapeDtypeStruct((grid, TBLK // 128, 128), jnp.int32),
    )(flat_x, x2, embt)
    return idx2d.reshape(-1)


def _sc_decode_body(emb_hbm, idx_hbm, x_hbm, q_hbm, part_hbm,
                    idx_v, rows_v, x_v, out_v, acc_v, sem):
    wid = lax.axis_index("s") * NC + lax.axis_index("c")
    base = wid * BPW
    pltpu.sync_copy(idx_hbm.at[pl.ds(base, BPW)], idx_v)
    pltpu.sync_copy(x_hbm.at[pl.ds(base, BPW)], x_v)
    copies = [
        pltpu.make_async_copy(
            emb_hbm.at[idx_v.at[pl.ds(j * 128, 128)]],
            rows_v.at[pl.ds(j * 128, 128)],
            sem,
        )
        for j in range(BPW // 128)
    ]
    for c in copies:
        c.start()
    for c in copies:
        c.wait()
    acc_v[...] = jnp.zeros((LANES,), jnp.float32)

    def body(i, carry):
        for h in range(D // LANES):
            s = pl.ds(h * LANES, LANES)
            q = rows_v[i, s]
            xv = x_v[i, s]
            qq = xv + (q - xv)
            r = qq - xv
            acc_v[...] += r * r
            out_v[i, s] = qq
        return carry

    lax.fori_loop(0, BPW, body, 0)
    pltpu.sync_copy(out_v, q_hbm.at[pl.ds(base, BPW)])
    pltpu.sync_copy(acc_v, part_hbm.at[wid])


@functools.cache
def _sc_decode():
    return pl.kernel(
        _sc_decode_body,
        out_type=(
            jax.ShapeDtypeStruct((NUM_TOKENS, D), jnp.float32),
            jax.ShapeDtypeStruct((NW, LANES), jnp.float32),
        ),
        mesh=plsc.VectorSubcoreMesh(
            core_axis_name="c", subcore_axis_name="s",
            num_cores=NC, num_subcores=NS),
        scratch_types=[
            pltpu.VMEM((BPW,), jnp.int32),
            pltpu.VMEM((BPW, 128), jnp.float32),
            pltpu.VMEM((BPW, D), jnp.float32),
            pltpu.VMEM((BPW, D), jnp.float32),
            pltpu.VMEM((LANES,), jnp.float32),
            pltpu.SemaphoreType.DMA,
        ],
    )


def kernel(x, embedding_weight):
    flat_x = x.reshape(-1, D)
    embt = embedding_weight.T
    x2 = jnp.sum(flat_x ** 2, axis=1, keepdims=True)
    idx = _tc_argmin(flat_x, x2, embt)
    emb_pad = jnp.pad(embedding_weight, ((0, 0), (0, 128 - D)))
    q, part = _sc_decode()(emb_pad, idx, flat_x)
    quantized = q.reshape(x.shape)
    diff = jnp.sum(part) / (NUM_TOKENS * D)
    return quantized, diff
